# baseline (speedup 1.0000x reference)
import functools

import jax
import jax.numpy as jnp
from jax import lax
from jax.experimental import pallas as pl
from jax.experimental.pallas import tpu as pltpu
from jax.experimental.pallas import tpu_sc as plsc

_EMBED_DIM = 32
_NUM_CORES = 2
_NUM_SUBCORES = 16
_NW = _NUM_CORES * _NUM_SUBCORES
_CHUNK = 1600
_L = 16


def _embed_body(x_hbm, table_hbm, out_hbm, idx_v, rows_v, sem_i, sem_g, sem_st):
    wid = lax.axis_index("s") * _NUM_CORES + lax.axis_index("c")
    n_total = x_hbm.shape[0]
    per_w = n_total // _NW
    n_chunks = per_w // _CHUNK
    base = wid * per_w

    pltpu.make_async_copy(x_hbm.at[pl.ds(base, per_w)], idx_v, sem_i).start()
    pltpu.make_async_copy(x_hbm.at[pl.ds(base, per_w)], idx_v, sem_i).wait()

    def remap(j, carry):
        v = idx_v[pl.ds(j * _L, _L)]
        q = (v & jnp.int32(~31)) | ((v & 7) << 2) | ((v >> 3) & 3)
        idx_v[pl.ds(j * _L, _L)] = q
        return carry

    lax.fori_loop(0, per_w // _L, remap, 0)

    st_dma = [None, None]
    g_dma = [None, None]

    def start_gather(i):
        b = i % 2
        g_dma[b] = pltpu.make_async_copy(
            table_hbm.at[idx_v.at[pl.ds(i * _CHUNK, _CHUNK)]],
            rows_v.at[b],
            sem_g.at[b],
        )
        g_dma[b].start()

    start_gather(0)
    for i in range(n_chunks):
        b = i % 2
        if i + 1 < n_chunks:
            start_gather(i + 1)
        g_dma[b].wait()
        if i >= 2:
            st_dma[b].wait()
        st_dma[b] = pltpu.make_async_copy(
            rows_v.at[b], out_hbm.at[pl.ds(base + i * _CHUNK, _CHUNK)], sem_st.at[b]
        )
        st_dma[b].start()
    st_dma[(n_chunks - 2) % 2].wait()
    st_dma[(n_chunks - 1) % 2].wait()


def _make_lookup(n_rows):
    mesh = plsc.VectorSubcoreMesh(core_axis_name="c", subcore_axis_name="s")
    per_w = n_rows // _NW
    return functools.partial(
        pl.kernel,
        mesh=mesh,
        out_type=jax.ShapeDtypeStruct((n_rows, _EMBED_DIM), jnp.float32),
        scratch_types=[
            pltpu.VMEM((per_w,), jnp.int32),
            pltpu.VMEM((2, _CHUNK, _EMBED_DIM), jnp.float32),
            pltpu.SemaphoreType.DMA,
            pltpu.SemaphoreType.DMA((2,)),
            pltpu.SemaphoreType.DMA((2,)),
        ],
        compiler_params=pltpu.CompilerParams(use_tc_tiling_on_sc=False),
    )(_embed_body)


@jax.jit
def kernel(x, table):
    b, h = x.shape
    flat = x.reshape(b * h).astype(jnp.int32)
    v = table.shape[0]
    t2 = (
        table.reshape(v // 32, 4, 8, _EMBED_DIM)
        .transpose(0, 2, 1, 3)
        .reshape(v, _EMBED_DIM)
    )
    out = _make_lookup(b * h)(flat, t2)
    return out.reshape(b, h, _EMBED_DIM)


# speedup vs baseline: 1.0127x; 1.0127x over previous
import functools

import jax
import jax.numpy as jnp
from jax import lax
from jax.experimental import pallas as pl
from jax.experimental.pallas import tpu as pltpu
from jax.experimental.pallas import tpu_sc as plsc

_D = 32
_NUM_CORES = 2
_NUM_SUBCORES = 16
_NW = _NUM_CORES * _NUM_SUBCORES
_HS = 8
_BL = 128
_SB = _HS * _BL
_L = 16


def _embed_body(xn_hbm, table_hbm, out_hbm, idx_v, rows_v, tv, sem_i, sem_g, sem_st):
    wid = lax.axis_index("s") * _NUM_CORES + lax.axis_index("c")
    n_sb = xn_hbm.shape[0] // _SB // _NW
    nbt = 32
    m0 = wid * n_sb

    iota = jax.lax.iota(jnp.int32, _L)
    zero16 = iota * 0

    idx_dma = [None, None]
    g_dma = [None, None]

    def start_idx(i):
        b = i % 2
        idx_dma[b] = pltpu.make_async_copy(
            xn_hbm.at[pl.ds((m0 + i) * _SB, _SB)], idx_v.at[b], sem_i.at[b]
        )
        idx_dma[b].start()

    def start_gather(i):
        b = i % 2
        g_dma[b] = pltpu.make_async_copy(
            table_hbm.at[idx_v.at[b]], rows_v.at[b], sem_g.at[b]
        )
        g_dma[b].start()

    start_idx(0)
    idx_dma[0].wait()
    start_gather(0)
    for i in range(n_sb):
        b = i % 2
        m = m0 + i
        ht = m // nbt
        bt = m % nbt
        if i + 1 < n_sb:
            start_idx(i + 1)
            idx_dma[(i + 1) % 2].wait()
        g_dma[b].wait()
        if i + 1 < n_sb:
            start_gather(i + 1)
        if i >= 1:
            pltpu.make_async_copy(tv, out_hbm.at[pl.ds(0, _HS * 4096)], sem_st).wait()

        rows = rows_v.at[b]

        def transpose_step(k, carry):
            hs = k // _D
            c = k % _D
            rb = hs * _BL
            db = hs * 4096 + c * _BL
            for j in range(_BL // _L):
                ridx = rb + j * _L + iota
                v = plsc.load_gather(rows, [ridx, zero16 + c])
                plsc.store_scatter(tv, [db + j * _L + iota], v)
            return carry

        lax.fori_loop(0, _HS * _D, transpose_step, 0)

        def store_step(s, carry):
            hs = s // 4
            ct = s % 4
            h = ht * _HS + hs
            off = ((h * 4 + ct) * nbt + bt) * 1024
            pltpu.make_async_copy(
                tv.at[pl.ds(s * 1024, 1024)], out_hbm.at[pl.ds(off, 1024)], sem_st
            ).start()
            return carry

        lax.fori_loop(0, _HS * 4, store_step, 0)
    pltpu.make_async_copy(tv, out_hbm.at[pl.ds(0, _HS * 4096)], sem_st).wait()


def _make_lookup(n_rows):
    mesh = plsc.VectorSubcoreMesh(core_axis_name="c", subcore_axis_name="s")
    return functools.partial(
        pl.kernel,
        mesh=mesh,
        out_type=jax.ShapeDtypeStruct((n_rows * _D,), jnp.float32),
        scratch_types=[
            pltpu.VMEM((2, _SB), jnp.int32),
            pltpu.VMEM((2, _SB, _D), jnp.float32),
            pltpu.VMEM((_HS * 4096,), jnp.float32),
            pltpu.SemaphoreType.DMA((2,)),
            pltpu.SemaphoreType.DMA((2,)),
            pltpu.SemaphoreType.DMA,
        ],
        compiler_params=pltpu.CompilerParams(
            use_tc_tiling_on_sc=False, needs_layout_passes=False
        ),
    )(_embed_body)


@jax.jit
def kernel(x, table):
    b, h = x.shape
    ht, bt = h // _HS, b // _BL
    xn = (
        x.astype(jnp.int32)
        .T.reshape(ht, _HS, bt, _BL)
        .transpose(0, 2, 1, 3)
        .reshape(b * h)
    )
    out5 = _make_lookup(b * h)(xn, table)
    return (
        out5.reshape(ht * _HS, _D // _HS, bt, _HS, _BL)
        .transpose(2, 4, 0, 1, 3)
        .reshape(b, h, _D)
    )


# speedup vs baseline: 1.5719x; 1.5522x over previous
import functools

import jax
import jax.numpy as jnp
from jax import lax
from jax.experimental import pallas as pl
from jax.experimental.pallas import tpu as pltpu
from jax.experimental.pallas import tpu_sc as plsc

_D = 32
_NUM_CORES = 2
_NUM_SUBCORES = 16
_NW = _NUM_CORES * _NUM_SUBCORES
_HS = 8
_BL = 128
_SB = _HS * _BL
_L = 16


def _embed_body(xn_hbm, table_hbm, out_hbm, idx_v, rows_v, tv, sem_i, sem_g, sem_st):
    wid = lax.axis_index("s") * _NUM_CORES + lax.axis_index("c")
    n_sb = xn_hbm.shape[0] // _SB // _NW
    nbt = 32
    m0 = wid * n_sb

    iota = jax.lax.iota(jnp.int32, _L)
    zero16 = iota * 0

    idx_dma = [None, None]
    g_dma = [None, None]

    def start_idx(i):
        b = i % 2
        idx_dma[b] = pltpu.make_async_copy(
            xn_hbm.at[pl.ds((m0 + i) * _SB, _SB)], idx_v.at[b], sem_i.at[b]
        )
        idx_dma[b].start()

    def start_gather(i):
        b = i % 2
        g_dma[b] = pltpu.make_async_copy(
            table_hbm.at[idx_v.at[b]], rows_v.at[b], sem_g.at[b]
        )
        g_dma[b].start()

    start_idx(0)
    idx_dma[0].wait()
    start_gather(0)
    for i in range(n_sb):
        b = i % 2
        m = m0 + i
        ht = m // nbt
        bt = m % nbt
        if i + 1 < n_sb:
            start_idx(i + 1)
            idx_dma[(i + 1) % 2].wait()
        g_dma[b].wait()
        if i + 1 < n_sb:
            start_gather(i + 1)
        if i >= 1:
            def drain_step(s, carry):
                pltpu.make_async_copy(
                    tv.at[pl.ds(s * 8, 8), pl.ds(0, _BL)], out_hbm.at[s], sem_st
                ).wait()
                return carry

            lax.fori_loop(0, _HS * 4, drain_step, 0)

        rows = rows_v.at[b]

        def transpose_step(k, carry):
            for u in range(4):
                r = k * 4 + u
                hs = r // _BL
                bl = r % _BL
                col = zero16 + bl
                v0 = rows[r, pl.ds(0, _L)]
                plsc.store_scatter(tv, [hs * _D + iota, col], v0)
                v1 = rows[r, pl.ds(_L, _L)]
                plsc.store_scatter(tv, [hs * _D + _L + iota, col], v1)
            return carry

        lax.fori_loop(0, _SB // 4, transpose_step, 0)

        def store_step(s, carry):
            hs = s // 4
            ct = s % 4
            h = ht * _HS + hs
            blk = (h * 4 + ct) * nbt + bt
            pltpu.make_async_copy(
                tv.at[pl.ds(s * 8, 8), pl.ds(0, _BL)], out_hbm.at[blk], sem_st
            ).start()
            return carry

        lax.fori_loop(0, _HS * 4, store_step, 0)

    def drain_last(s, carry):
        pltpu.make_async_copy(tv.at[pl.ds(s * 8, 8), pl.ds(0, _BL)], out_hbm.at[s], sem_st).wait()
        return carry

    lax.fori_loop(0, _HS * 4, drain_last, 0)


def _make_lookup(n_rows):
    mesh = plsc.VectorSubcoreMesh(core_axis_name="c", subcore_axis_name="s")
    return functools.partial(
        pl.kernel,
        mesh=mesh,
        out_type=jax.ShapeDtypeStruct((n_rows * _D // 1024, _HS, _BL), jnp.float32),
        scratch_types=[
            pltpu.VMEM((2, _SB), jnp.int32),
            pltpu.VMEM((2, _SB, _D), jnp.float32),
            pltpu.VMEM((_HS * _D, 129), jnp.float32),
            pltpu.SemaphoreType.DMA((2,)),
            pltpu.SemaphoreType.DMA((2,)),
            pltpu.SemaphoreType.DMA,
        ],
        compiler_params=pltpu.CompilerParams(
            use_tc_tiling_on_sc=False, needs_layout_passes=False
        ),
    )(_embed_body)


@jax.jit
def kernel(x, table):
    b, h = x.shape
    ht, bt = h // _HS, b // _BL
    xn = (
        x.astype(jnp.int32)
        .T.reshape(ht, _HS, bt, _BL)
        .transpose(0, 2, 1, 3)
        .reshape(b * h)
    )
    out5 = _make_lookup(b * h)(xn, table)
    return (
        out5.reshape(ht * _HS, _D // _HS, bt, _HS, _BL)
        .transpose(2, 4, 0, 1, 3)
        .reshape(b, h, _D)
    )
